# Initial kernel scaffold; baseline (speedup 1.0000x reference)
import jax
import jax.numpy as jnp
from jax.experimental import pallas as pl


def kernel(x, e, hyperedge_index, node_types, hyperedge_types, ln_gamma, ln_beta, W, b):
    raise NotImplementedError("write your pallas kernel here")


# speedup vs baseline: 136.5655x; 136.5655x over previous
import functools

import jax
import jax.numpy as jnp
from jax import lax
from jax.experimental import pallas as pl
from jax.experimental.pallas import tpu as pltpu
from jax.experimental.pallas import tpu_sc as plsc

_D = 5
_HID = 64
_OUT = _D * (_D - 1) // 2

_B = 2560
_R = _B // 128


def _pool_body(x_ref, e_ref, xm_ref, em_ref):
    xv = x_ref[...]
    ev = e_ref[...]
    xs = xv[:, 0:_HID]
    es = ev[:, 0:_HID]
    for d in range(1, _D):
        xs = xs + xv[:, d * _HID:(d + 1) * _HID]
        es = es + ev[:, d * _HID:(d + 1) * _HID]
    xm_ref[...] = xs * (1.0 / _D)
    em_ref[...] = es * (1.0 / _D)


def _pool(x2, e2):
    n = x2.shape[0]
    blk = 2000
    return pl.pallas_call(
        _pool_body,
        grid=(n // blk,),
        in_specs=[
            pl.BlockSpec((blk, _D * _HID), lambda i: (i, jnp.int32(0))),
            pl.BlockSpec((blk, _D * _HID), lambda i: (i, jnp.int32(0))),
        ],
        out_specs=[
            pl.BlockSpec((blk, _HID), lambda i: (i, jnp.int32(0))),
            pl.BlockSpec((blk, _HID), lambda i: (i, jnp.int32(0))),
        ],
        out_shape=[
            jax.ShapeDtypeStruct((n, _HID), jnp.float32),
            jax.ShapeDtypeStruct((n, _HID), jnp.float32),
        ],
    )(x2, e2)


def _make_gather(nnz):
    info = plsc.get_sparse_core_info()
    nc, ns = info.num_cores, info.num_subcores
    nw = nc * ns
    b_per_w = nnz // nw
    chunk = 1000
    nchunks = b_per_w // chunk
    mesh = plsc.VectorSubcoreMesh(core_axis_name="c", subcore_axis_name="s")

    @functools.partial(
        pl.kernel,
        out_type=[jax.ShapeDtypeStruct((nnz, _HID), jnp.float32)] * 2,
        mesh=mesh,
        scratch_types=[
            pltpu.VMEM((chunk,), jnp.int32),
            pltpu.VMEM((chunk, _HID), jnp.float32),
            pltpu.SemaphoreType.DMA,
        ],
        compiler_params=pltpu.CompilerParams(use_tc_tiling_on_sc=False),
    )
    def gat(xm_hbm, em_hbm, row_hbm, col_hbm, xr_hbm, ec_hbm, idx_v, rows_v, sem):
        wid = lax.axis_index("s") * nc + lax.axis_index("c")
        base = wid * b_per_w

        for j in range(nchunks):
            off = base + j * chunk
            pltpu.sync_copy(row_hbm.at[pl.ds(off, chunk)], idx_v)
            pltpu.async_copy(xm_hbm.at[idx_v], rows_v, sem).wait()
            pltpu.sync_copy(rows_v, xr_hbm.at[pl.ds(off, chunk)])
            pltpu.sync_copy(col_hbm.at[pl.ds(off, chunk)], idx_v)
            pltpu.async_copy(em_hbm.at[idx_v], rows_v, sem).wait()
            pltpu.sync_copy(rows_v, ec_hbm.at[pl.ds(off, chunk)])

    return gat


def _to_rl(row_1b):
    return row_1b.reshape(_R, 128)


def _main_body(xr_ref, ec_ref, row_ref, col_ref, wt_ref, cv_ref,
               at_ref, hi_ref):
    f32 = jnp.float32
    xr = xr_ref[...]
    ec = ec_ref[...]
    hh = jnp.concatenate([xr, ec, xr * xr, ec * ec], axis=1)
    s = lax.dot_general(wt_ref[...], hh, (((1,), (1,)), ((), ())),
                        preferred_element_type=f32)
    mu = s[10:11, :]
    m2 = s[11:12, :]
    inv = lax.rsqrt(m2 - mu * mu + 1e-5)
    z = s[0:_OUT, :] * inv + cv_ref[...]
    sg = 1.0 / (1.0 + jnp.exp(-z))
    p = [_to_rl(sg[k:k + 1, :]) for k in range(_OUT)]

    v0 = [None, p[0], p[1], p[3], p[6]]
    d0 = 1.0 + p[0] * p[0] + p[1] * p[1] + p[3] * p[3] + p[6] * p[6]
    s0 = 2.0 / d0
    u = [s0] + [s0 * v0[a] for a in range(1, _D)]
    q = [[None] * _D for _ in range(_D)]
    for a in range(_D):
        q[a][0] = (1.0 - u[0]) if a == 0 else (-u[a])
        for b in range(1, _D):
            if a == b:
                q[a][b] = 1.0 - u[a] * v0[b]
            else:
                q[a][b] = -(u[a] * v0[b])

    for t, pars in ((1, (p[2], p[4], p[7])), (2, (p[5], p[8])), (3, (p[9],))):
        d = 1.0
        for a_p in pars:
            d = d + a_p * a_p
        st = 2.0 / d
        for a in range(_D):
            w = q[a][t]
            for m, a_p in enumerate(pars):
                w = w + q[a][t + 1 + m] * a_p
            sw = st * w
            q[a][t] = q[a][t] - sw
            for m, a_p in enumerate(pars):
                q[a][t + 1 + m] = q[a][t + 1 + m] - sw * a_p
    for a in range(_D):
        q[a][_D - 1] = -q[a][_D - 1]

    rows = [q[a][b].reshape(1, _B) for a in range(_D) for b in range(_D)]
    at_ref[...] = jnp.concatenate(rows, axis=0)

    l = lax.broadcasted_iota(jnp.int32, (_B, 2 * _D * _D), 1)
    k = lax.shift_right_logical(l, jnp.int32(1))
    i = lax.shift_right_logical(k * 52, jnp.int32(8))
    j = k - 5 * i
    low = (l & 1) == 0
    zero = jnp.zeros((), jnp.int32)
    rowv = row_ref[...]
    colv = col_ref[...]
    hi_ref[0] = jnp.where(low, 5 * rowv + i, zero)
    hi_ref[1] = jnp.where(low, 5 * colv + j, zero)


def _main(xr, ec, row2, col2, wt, cv):
    nnz = xr.shape[0]
    nb = nnz // _B
    grid = (nb,)
    return pl.pallas_call(
        _main_body,
        grid=grid,
        in_specs=[
            pl.BlockSpec((_B, _HID), lambda i: (i, jnp.int32(0))),
            pl.BlockSpec((_B, _HID), lambda i: (i, jnp.int32(0))),
            pl.BlockSpec((_B, 1), lambda i: (i, jnp.int32(0))),
            pl.BlockSpec((_B, 1), lambda i: (i, jnp.int32(0))),
            pl.BlockSpec((16, 4 * _HID), lambda i: (jnp.int32(0), jnp.int32(0))),
            pl.BlockSpec((_OUT, 1), lambda i: (jnp.int32(0), jnp.int32(0))),
        ],
        out_specs=[
            pl.BlockSpec((_D * _D, _B), lambda i: (jnp.int32(0), i)),
            pl.BlockSpec((2, _B, 2 * _D * _D), lambda i: (jnp.int32(0), i, jnp.int32(0))),
        ],
        out_shape=[
            jax.ShapeDtypeStruct((_D * _D, nnz), jnp.float32),
            jax.ShapeDtypeStruct((2, nnz, 2 * _D * _D), jnp.int32),
        ],
    )(xr, ec, row2, col2, wt, cv)


def kernel(x, e, hyperedge_index, node_types, hyperedge_types,
           ln_gamma, ln_beta, W, b):
    n_nodes = x.shape[0] // _D
    n_edges = e.shape[0] // _D
    nnz = hyperedge_index.shape[1]

    x2 = x.reshape(n_nodes, _D * _HID)
    e2 = e.reshape(n_edges, _D * _HID)
    xm, em = _pool(x2, e2)

    row32 = hyperedge_index[0].astype(jnp.int32)
    col32 = hyperedge_index[1].astype(jnp.int32)
    xr, ec = _make_gather(nnz)(xm, em, row32, col32)

    gw = W.astype(jnp.float32) * ln_gamma.astype(jnp.float32)[:, None]
    wc = gw - jnp.sum(gw, axis=0)[None, :] * (1.0 / (2 * _HID))
    wt = jnp.zeros((16, 4 * _HID), jnp.float32)
    wt = wt.at[0:_OUT, 0:2 * _HID].set(wc.T)
    wt = wt.at[10, 0:2 * _HID].set(1.0 / (2 * _HID))
    wt = wt.at[11, 2 * _HID:4 * _HID].set(1.0 / (2 * _HID))
    cv = (ln_beta.astype(jnp.float32) @ W.astype(jnp.float32)
          + b.astype(jnp.float32)).reshape(_OUT, 1)

    atT, hi = _main(xr, ec, row32.reshape(nnz, 1), col32.reshape(nnz, 1),
                    wt, cv)

    attrs = atT.T.reshape(-1).astype(jnp.float64)
    h64 = lax.bitcast_convert_type(
        hi.reshape(2, nnz, _D * _D, 2), jnp.int64)
    h_index = h64.reshape(2, nnz * _D * _D)
    return h_index, attrs


# speedup vs baseline: 149.5220x; 1.0949x over previous
import functools

import jax
import jax.numpy as jnp
from jax import lax
from jax.experimental import pallas as pl
from jax.experimental.pallas import tpu as pltpu
from jax.experimental.pallas import tpu_sc as plsc

_D = 5
_HID = 64
_OUT = _D * (_D - 1) // 2

_B = 2560
_R = _B // 128


def _pool_body(x_ref, e_ref, xm_ref, em_ref):
    xv = x_ref[...]
    ev = e_ref[...]
    xs = xv[:, 0:_HID]
    es = ev[:, 0:_HID]
    for d in range(1, _D):
        xs = xs + xv[:, d * _HID:(d + 1) * _HID]
        es = es + ev[:, d * _HID:(d + 1) * _HID]
    xm_ref[...] = xs * (1.0 / _D)
    em_ref[...] = es * (1.0 / _D)


def _pool(x2, e2):
    n = x2.shape[0]
    blk = 2000
    return pl.pallas_call(
        _pool_body,
        grid=(n // blk,),
        in_specs=[
            pl.BlockSpec((blk, _D * _HID), lambda i: (i, jnp.int32(0))),
            pl.BlockSpec((blk, _D * _HID), lambda i: (i, jnp.int32(0))),
        ],
        out_specs=[
            pl.BlockSpec((blk, _HID), lambda i: (i, jnp.int32(0))),
            pl.BlockSpec((blk, _HID), lambda i: (i, jnp.int32(0))),
        ],
        out_shape=[
            jax.ShapeDtypeStruct((n, _HID), jnp.float32),
            jax.ShapeDtypeStruct((n, _HID), jnp.float32),
        ],
    )(x2, e2)


def _make_gather(nnz):
    info = plsc.get_sparse_core_info()
    nc, ns = info.num_cores, info.num_subcores
    nw = nc * ns
    b_per_w = nnz // nw
    chunk = 1000
    nchunks = b_per_w // chunk
    mesh = plsc.VectorSubcoreMesh(core_axis_name="c", subcore_axis_name="s")

    @functools.partial(
        pl.kernel,
        out_type=[jax.ShapeDtypeStruct((nnz, _HID), jnp.float32)] * 2,
        mesh=mesh,
        scratch_types=[
            pltpu.VMEM((chunk,), jnp.int32),
            pltpu.VMEM((chunk, _HID), jnp.float32),
            pltpu.SemaphoreType.DMA,
        ],
        compiler_params=pltpu.CompilerParams(use_tc_tiling_on_sc=False),
    )
    def gat(xm_hbm, em_hbm, row_hbm, col_hbm, xr_hbm, ec_hbm, idx_v, rows_v, sem):
        wid = lax.axis_index("s") * nc + lax.axis_index("c")
        base = wid * b_per_w

        for j in range(nchunks):
            off = base + j * chunk
            pltpu.sync_copy(row_hbm.at[pl.ds(off, chunk)], idx_v)
            pltpu.async_copy(xm_hbm.at[idx_v], rows_v, sem).wait()
            pltpu.sync_copy(rows_v, xr_hbm.at[pl.ds(off, chunk)])
            pltpu.sync_copy(col_hbm.at[pl.ds(off, chunk)], idx_v)
            pltpu.async_copy(em_hbm.at[idx_v], rows_v, sem).wait()
            pltpu.sync_copy(rows_v, ec_hbm.at[pl.ds(off, chunk)])

    return gat


def _to_rl(row_1b):
    return row_1b.reshape(_R, 128)


def _main_body(xr_ref, ec_ref, row_ref, col_ref, wt_ref, cv_ref,
               at_ref, hi_ref):
    f32 = jnp.float32
    xr = xr_ref[...]
    ec = ec_ref[...]
    hh = jnp.concatenate([xr, ec, xr * xr, ec * ec], axis=1)
    s = lax.dot_general(wt_ref[...], hh, (((1,), (1,)), ((), ())),
                        preferred_element_type=f32)
    mu = s[10:11, :]
    m2 = s[11:12, :]
    inv = lax.rsqrt(m2 - mu * mu + 1e-5)
    z = s[0:_OUT, :] * inv + cv_ref[...]
    sg = 1.0 / (1.0 + jnp.exp(-z))
    p = [_to_rl(sg[k:k + 1, :]) for k in range(_OUT)]

    v0 = [None, p[0], p[1], p[3], p[6]]
    d0 = 1.0 + p[0] * p[0] + p[1] * p[1] + p[3] * p[3] + p[6] * p[6]
    s0 = 2.0 / d0
    u = [s0] + [s0 * v0[a] for a in range(1, _D)]
    q = [[None] * _D for _ in range(_D)]
    for a in range(_D):
        q[a][0] = (1.0 - u[0]) if a == 0 else (-u[a])
        for b in range(1, _D):
            if a == b:
                q[a][b] = 1.0 - u[a] * v0[b]
            else:
                q[a][b] = -(u[a] * v0[b])

    for t, pars in ((1, (p[2], p[4], p[7])), (2, (p[5], p[8])), (3, (p[9],))):
        d = 1.0
        for a_p in pars:
            d = d + a_p * a_p
        st = 2.0 / d
        for a in range(_D):
            w = q[a][t]
            for m, a_p in enumerate(pars):
                w = w + q[a][t + 1 + m] * a_p
            sw = st * w
            q[a][t] = q[a][t] - sw
            for m, a_p in enumerate(pars):
                q[a][t + 1 + m] = q[a][t + 1 + m] - sw * a_p
    for a in range(_D):
        q[a][_D - 1] = -q[a][_D - 1]

    rows = [q[a][b].reshape(1, _B) for a in range(_D) for b in range(_D)]
    at_ref[...] = jnp.concatenate(rows, axis=0)

    k = lax.broadcasted_iota(jnp.int32, (_B, _D * _D), 1)
    i = lax.shift_right_logical(k * 52, jnp.int32(8))
    j = k - 5 * i
    rowv = row_ref[...]
    colv = col_ref[...]
    hi_ref[0] = 5 * rowv + i
    hi_ref[1] = 5 * colv + j


def _main(xr, ec, row2, col2, wt, cv):
    nnz = xr.shape[0]
    nb = nnz // _B
    grid = (nb,)
    return pl.pallas_call(
        _main_body,
        grid=grid,
        in_specs=[
            pl.BlockSpec((_B, _HID), lambda i: (i, jnp.int32(0))),
            pl.BlockSpec((_B, _HID), lambda i: (i, jnp.int32(0))),
            pl.BlockSpec((_B, 1), lambda i: (i, jnp.int32(0))),
            pl.BlockSpec((_B, 1), lambda i: (i, jnp.int32(0))),
            pl.BlockSpec((16, 4 * _HID), lambda i: (jnp.int32(0), jnp.int32(0))),
            pl.BlockSpec((_OUT, 1), lambda i: (jnp.int32(0), jnp.int32(0))),
        ],
        out_specs=[
            pl.BlockSpec((_D * _D, _B), lambda i: (jnp.int32(0), i)),
            pl.BlockSpec((2, _B, _D * _D), lambda i: (jnp.int32(0), i, jnp.int32(0))),
        ],
        out_shape=[
            jax.ShapeDtypeStruct((_D * _D, nnz), jnp.float32),
            jax.ShapeDtypeStruct((2, nnz, _D * _D), jnp.int32),
        ],
    )(xr, ec, row2, col2, wt, cv)


def kernel(x, e, hyperedge_index, node_types, hyperedge_types,
           ln_gamma, ln_beta, W, b):
    n_nodes = x.shape[0] // _D
    n_edges = e.shape[0] // _D
    nnz = hyperedge_index.shape[1]

    x2 = x.reshape(n_nodes, _D * _HID)
    e2 = e.reshape(n_edges, _D * _HID)
    xm, em = _pool(x2, e2)

    row32 = hyperedge_index[0].astype(jnp.int32)
    col32 = hyperedge_index[1].astype(jnp.int32)
    xr, ec = _make_gather(nnz)(xm, em, row32, col32)

    gw = W.astype(jnp.float32) * ln_gamma.astype(jnp.float32)[:, None]
    wc = gw - jnp.sum(gw, axis=0)[None, :] * (1.0 / (2 * _HID))
    wt = jnp.zeros((16, 4 * _HID), jnp.float32)
    wt = wt.at[0:_OUT, 0:2 * _HID].set(wc.T)
    wt = wt.at[10, 0:2 * _HID].set(1.0 / (2 * _HID))
    wt = wt.at[11, 2 * _HID:4 * _HID].set(1.0 / (2 * _HID))
    cv = (ln_beta.astype(jnp.float32) @ W.astype(jnp.float32)
          + b.astype(jnp.float32)).reshape(_OUT, 1)

    atT, hi = _main(xr, ec, row32.reshape(nnz, 1), col32.reshape(nnz, 1),
                    wt, cv)

    attrs = atT.T.reshape(-1).astype(jnp.float64)
    h_index = hi.reshape(2, nnz * _D * _D).astype(jnp.int64)
    return h_index, attrs


# speedup vs baseline: 238.9758x; 1.5983x over previous
import functools

import jax
import jax.numpy as jnp
from jax import lax
from jax.experimental import pallas as pl
from jax.experimental.pallas import tpu as pltpu
from jax.experimental.pallas import tpu_sc as plsc

_D = 5
_HID = 64
_OUT = _D * (_D - 1) // 2

_B = 2560
_R = _B // 128


def _pool_body(x_ref, e_ref, xm_ref, em_ref):
    xv = x_ref[...]
    ev = e_ref[...]
    xs = xv[:, 0:_HID]
    es = ev[:, 0:_HID]
    for d in range(1, _D):
        xs = xs + xv[:, d * _HID:(d + 1) * _HID]
        es = es + ev[:, d * _HID:(d + 1) * _HID]
    xm_ref[...] = xs * (1.0 / _D)
    em_ref[...] = es * (1.0 / _D)


def _pool(x2, e2):
    n = x2.shape[0]
    blk = 2000
    return pl.pallas_call(
        _pool_body,
        grid=(n // blk,),
        in_specs=[
            pl.BlockSpec((blk, _D * _HID), lambda i: (i, jnp.int32(0))),
            pl.BlockSpec((blk, _D * _HID), lambda i: (i, jnp.int32(0))),
        ],
        out_specs=[
            pl.BlockSpec((blk, _HID), lambda i: (i, jnp.int32(0))),
            pl.BlockSpec((blk, _HID), lambda i: (i, jnp.int32(0))),
        ],
        out_shape=[
            jax.ShapeDtypeStruct((n, _HID), jnp.float32),
            jax.ShapeDtypeStruct((n, _HID), jnp.float32),
        ],
    )(x2, e2)


def _make_gather(nnz):
    info = plsc.get_sparse_core_info()
    nc, ns = info.num_cores, info.num_subcores
    nw = nc * ns
    b_per_w = nnz // nw
    chunk = 1000
    nchunks = b_per_w // chunk
    mesh = plsc.VectorSubcoreMesh(core_axis_name="c", subcore_axis_name="s")

    ef = _D * _D
    exp = chunk * ef
    eit = (exp + 15) // 16 + 1

    @functools.partial(
        pl.kernel,
        out_type=[
            jax.ShapeDtypeStruct((nnz, _HID), jnp.float32),
            jax.ShapeDtypeStruct((nnz, _HID), jnp.float32),
            jax.ShapeDtypeStruct((2, nnz * ef), jnp.int32),
        ],
        mesh=mesh,
        scratch_types=[
            pltpu.VMEM((chunk + 16,), jnp.int32),
            pltpu.VMEM((chunk + 16,), jnp.int32),
            pltpu.VMEM((chunk, _HID), jnp.float32),
            pltpu.VMEM((16 * eit,), jnp.int32),
            pltpu.VMEM((16 * eit,), jnp.int32),
            pltpu.SemaphoreType.DMA,
        ],
        compiler_params=pltpu.CompilerParams(use_tc_tiling_on_sc=False, needs_layout_passes=False),
    )
    def gat(xm_hbm, em_hbm, row_hbm, col_hbm, xr_hbm, ec_hbm, hx_hbm,
            idxr_v, idxc_v, rows_v, outr_v, outc_v, sem):
        wid = lax.axis_index("s") * nc + lax.axis_index("c")
        base = wid * b_per_w
        lane = lax.iota(jnp.int32, 16)

        for j in range(nchunks):
            off = base + j * chunk
            pltpu.sync_copy(row_hbm.at[pl.ds(off, chunk)],
                            idxr_v.at[pl.ds(0, chunk)])
            pltpu.sync_copy(col_hbm.at[pl.ds(off, chunk)],
                            idxc_v.at[pl.ds(0, chunk)])
            pltpu.async_copy(xm_hbm.at[idxr_v.at[pl.ds(0, chunk)]],
                             rows_v, sem).wait()
            pltpu.sync_copy(rows_v, xr_hbm.at[pl.ds(off, chunk)])
            pltpu.async_copy(em_hbm.at[idxc_v.at[pl.ds(0, chunk)]],
                             rows_v, sem).wait()
            pltpu.sync_copy(rows_v, ec_hbm.at[pl.ds(off, chunk)])

            def ebody(t, carry):
                f = t * 16 + lane
                n = lax.shift_right_logical(f * 5243, jnp.int32(17))
                k = f - 25 * n
                ii = lax.shift_right_logical(k * 52, jnp.int32(8))
                jj = k - 5 * ii
                gr = plsc.load_gather(idxr_v, [n])
                gc = plsc.load_gather(idxc_v, [n])
                outr_v[pl.ds(t * 16, 16)] = 5 * gr + ii
                outc_v[pl.ds(t * 16, 16)] = 5 * gc + jj
                return carry

            lax.fori_loop(jnp.int32(0), jnp.int32(eit), ebody, jnp.int32(0))
            pltpu.sync_copy(outr_v.at[pl.ds(0, exp)],
                            hx_hbm.at[jnp.int32(0), pl.ds(off * ef, exp)])
            pltpu.sync_copy(outc_v.at[pl.ds(0, exp)],
                            hx_hbm.at[jnp.int32(1), pl.ds(off * ef, exp)])

    return gat


def _to_rl(row_1b):
    return row_1b.reshape(_R, 128)


def _main_body(xr_ref, ec_ref, wt_ref, cv_ref, at_ref):
    f32 = jnp.float32
    xr = xr_ref[...]
    ec = ec_ref[...]
    hh = jnp.concatenate([xr, ec, xr * xr, ec * ec], axis=1)
    s = lax.dot_general(wt_ref[...], hh, (((1,), (1,)), ((), ())),
                        preferred_element_type=f32)
    mu = s[10:11, :]
    m2 = s[11:12, :]
    inv = lax.rsqrt(m2 - mu * mu + 1e-5)
    z = s[0:_OUT, :] * inv + cv_ref[...]
    sg = 1.0 / (1.0 + jnp.exp(-z))
    p = [_to_rl(sg[k:k + 1, :]) for k in range(_OUT)]

    v0 = [None, p[0], p[1], p[3], p[6]]
    d0 = 1.0 + p[0] * p[0] + p[1] * p[1] + p[3] * p[3] + p[6] * p[6]
    s0 = 2.0 / d0
    u = [s0] + [s0 * v0[a] for a in range(1, _D)]
    q = [[None] * _D for _ in range(_D)]
    for a in range(_D):
        q[a][0] = (1.0 - u[0]) if a == 0 else (-u[a])
        for b in range(1, _D):
            if a == b:
                q[a][b] = 1.0 - u[a] * v0[b]
            else:
                q[a][b] = -(u[a] * v0[b])

    for t, pars in ((1, (p[2], p[4], p[7])), (2, (p[5], p[8])), (3, (p[9],))):
        d = 1.0
        for a_p in pars:
            d = d + a_p * a_p
        st = 2.0 / d
        for a in range(_D):
            w = q[a][t]
            for m, a_p in enumerate(pars):
                w = w + q[a][t + 1 + m] * a_p
            sw = st * w
            q[a][t] = q[a][t] - sw
            for m, a_p in enumerate(pars):
                q[a][t + 1 + m] = q[a][t + 1 + m] - sw * a_p
    for a in range(_D):
        q[a][_D - 1] = -q[a][_D - 1]

    rows = [q[a][b].reshape(1, _B) for a in range(_D) for b in range(_D)]
    at_ref[...] = jnp.concatenate(rows, axis=0)


def _main(xr, ec, wt, cv):
    nnz = xr.shape[0]
    nb = nnz // _B
    return pl.pallas_call(
        _main_body,
        grid=(nb,),
        in_specs=[
            pl.BlockSpec((_B, _HID), lambda i: (i, jnp.int32(0))),
            pl.BlockSpec((_B, _HID), lambda i: (i, jnp.int32(0))),
            pl.BlockSpec((16, 4 * _HID), lambda i: (jnp.int32(0), jnp.int32(0))),
            pl.BlockSpec((_OUT, 1), lambda i: (jnp.int32(0), jnp.int32(0))),
        ],
        out_specs=pl.BlockSpec((_D * _D, _B), lambda i: (jnp.int32(0), i)),
        out_shape=jax.ShapeDtypeStruct((_D * _D, nnz), jnp.float32),
    )(xr, ec, wt, cv)


def kernel(x, e, hyperedge_index, node_types, hyperedge_types,
           ln_gamma, ln_beta, W, b):
    n_nodes = x.shape[0] // _D
    n_edges = e.shape[0] // _D
    nnz = hyperedge_index.shape[1]

    x2 = x.reshape(n_nodes, _D * _HID)
    e2 = e.reshape(n_edges, _D * _HID)
    xm, em = _pool(x2, e2)

    row32 = hyperedge_index[0].astype(jnp.int32)
    col32 = hyperedge_index[1].astype(jnp.int32)
    xr, ec, hx = _make_gather(nnz)(xm, em, row32, col32)

    gw = W.astype(jnp.float32) * ln_gamma.astype(jnp.float32)[:, None]
    wc = gw - jnp.sum(gw, axis=0)[None, :] * (1.0 / (2 * _HID))
    wt = jnp.zeros((16, 4 * _HID), jnp.float32)
    wt = wt.at[0:_OUT, 0:2 * _HID].set(wc.T)
    wt = wt.at[10, 0:2 * _HID].set(1.0 / (2 * _HID))
    wt = wt.at[11, 2 * _HID:4 * _HID].set(1.0 / (2 * _HID))
    cv = (ln_beta.astype(jnp.float32) @ W.astype(jnp.float32)
          + b.astype(jnp.float32)).reshape(_OUT, 1)

    atT = _main(xr, ec, wt, cv)

    attrs = atT.T.reshape(-1).astype(jnp.float64)
    h_index = hx.astype(jnp.int64)
    return h_index, attrs


# speedup vs baseline: 368.1517x; 1.5405x over previous
import functools

import jax
import jax.numpy as jnp
from jax import lax
from jax.experimental import pallas as pl
from jax.experimental.pallas import tpu as pltpu
from jax.experimental.pallas import tpu_sc as plsc

_D = 5
_HID = 64
_OUT = _D * (_D - 1) // 2

_B = 2560
_R = _B // 128


def _pool_body(x_ref, e_ref, xm_ref, em_ref):
    xv = x_ref[...]
    ev = e_ref[...]
    xs = xv[:, 0:_HID]
    es = ev[:, 0:_HID]
    for d in range(1, _D):
        xs = xs + xv[:, d * _HID:(d + 1) * _HID]
        es = es + ev[:, d * _HID:(d + 1) * _HID]
    xm_ref[...] = xs * (1.0 / _D)
    em_ref[...] = es * (1.0 / _D)


def _pool(x2, e2):
    n = x2.shape[0]
    blk = 2000
    return pl.pallas_call(
        _pool_body,
        grid=(n // blk,),
        in_specs=[
            pl.BlockSpec((blk, _D * _HID), lambda i: (i, jnp.int32(0))),
            pl.BlockSpec((blk, _D * _HID), lambda i: (i, jnp.int32(0))),
        ],
        out_specs=[
            pl.BlockSpec((blk, _HID), lambda i: (i, jnp.int32(0))),
            pl.BlockSpec((blk, _HID), lambda i: (i, jnp.int32(0))),
        ],
        out_shape=[
            jax.ShapeDtypeStruct((n, _HID), jnp.float32),
            jax.ShapeDtypeStruct((n, _HID), jnp.float32),
        ],
    )(x2, e2)


def _make_gather(nnz):
    info = plsc.get_sparse_core_info()
    nc, ns = info.num_cores, info.num_subcores
    nw = nc * ns
    b_per_w = nnz // nw
    chunk = 1000
    nchunks = b_per_w // chunk
    mesh = plsc.VectorSubcoreMesh(core_axis_name="c", subcore_axis_name="s")

    ef = _D * _D
    npb = 512
    gpb = npb * ef
    nbat = nnz // npb
    bat_per_w = (nbat + nw - 1) // nw

    @functools.partial(
        pl.kernel,
        out_type=[
            jax.ShapeDtypeStruct((nnz, _HID), jnp.float32),
            jax.ShapeDtypeStruct((nnz, _HID), jnp.float32),
            jax.ShapeDtypeStruct((2 * nnz * ef,), jnp.int32),
        ],
        mesh=mesh,
        scratch_types=[
            pltpu.VMEM((chunk + 16,), jnp.int32),
            pltpu.VMEM((chunk + 16,), jnp.int32),
            pltpu.VMEM((chunk, _HID), jnp.float32),
            pltpu.VMEM((npb + 16,), jnp.int32),
            pltpu.VMEM((npb + 16,), jnp.int32),
            pltpu.VMEM((2 * gpb,), jnp.int32),
            pltpu.SemaphoreType.DMA,
        ],
        compiler_params=pltpu.CompilerParams(use_tc_tiling_on_sc=False, needs_layout_passes=False),
    )
    def gat(xm_hbm, em_hbm, row_hbm, col_hbm, xr_hbm, ec_hbm, hx_hbm,
            idxr_v, idxc_v, rows_v, eidr_v, eidc_v, out_v, sem):
        wid = lax.axis_index("s") * nc + lax.axis_index("c")
        base = wid * b_per_w
        lane = lax.iota(jnp.int32, 16)

        for j in range(nchunks):
            off = base + j * chunk
            pltpu.sync_copy(row_hbm.at[pl.ds(off, chunk)],
                            idxr_v.at[pl.ds(0, chunk)])
            pltpu.sync_copy(col_hbm.at[pl.ds(off, chunk)],
                            idxc_v.at[pl.ds(0, chunk)])
            pltpu.async_copy(xm_hbm.at[idxr_v.at[pl.ds(0, chunk)]],
                             rows_v, sem).wait()
            pltpu.sync_copy(rows_v, xr_hbm.at[pl.ds(off, chunk)])
            pltpu.async_copy(em_hbm.at[idxc_v.at[pl.ds(0, chunk)]],
                             rows_v, sem).wait()
            pltpu.sync_copy(rows_v, ec_hbm.at[pl.ds(off, chunk)])

        def bat(t, carry):
            b = wid + nw * t

            @pl.when(b < nbat)
            def _():
                n0 = b * npb
                pltpu.sync_copy(row_hbm.at[pl.ds(n0, npb)],
                                eidr_v.at[pl.ds(0, npb)])
                pltpu.sync_copy(col_hbm.at[pl.ds(n0, npb)],
                                eidc_v.at[pl.ds(0, npb)])

                def estep(s, c2):
                    l = s * 16 + lane
                    n = lax.shift_right_logical(l * 5243, jnp.int32(17))
                    k = l - 25 * n
                    ii = lax.shift_right_logical(k * 52, jnp.int32(8))
                    jj = k - 5 * ii
                    gr = plsc.load_gather(eidr_v, [n])
                    gc = plsc.load_gather(eidc_v, [n])
                    doff = 256 * lax.shift_right_logical(s, jnp.int32(3)) \
                        + 16 * (s & 7)
                    out_v[pl.ds(doff, 16)] = 5 * gr + ii
                    out_v[pl.ds(doff + 128, 16)] = 5 * gc + jj
                    return c2

                lax.fori_loop(jnp.int32(0), jnp.int32(gpb // 16), estep,
                              jnp.int32(0))
                pltpu.sync_copy(out_v, hx_hbm.at[pl.ds(b * 2 * gpb, 2 * gpb)])

            return carry

        lax.fori_loop(jnp.int32(0), jnp.int32(bat_per_w), bat, jnp.int32(0))

    return gat


def _to_rl(row_1b):
    return row_1b.reshape(_R, 128)


def _main_body(xr_ref, ec_ref, wt_ref, cv_ref, at_ref):
    f32 = jnp.float32
    xr = xr_ref[...]
    ec = ec_ref[...]
    hh = jnp.concatenate([xr, ec, xr * xr, ec * ec], axis=1)
    s = lax.dot_general(wt_ref[...], hh, (((1,), (1,)), ((), ())),
                        preferred_element_type=f32)
    mu = s[10:11, :]
    m2 = s[11:12, :]
    inv = lax.rsqrt(m2 - mu * mu + 1e-5)
    z = s[0:_OUT, :] * inv + cv_ref[...]
    sg = 1.0 / (1.0 + jnp.exp(-z))
    p = [_to_rl(sg[k:k + 1, :]) for k in range(_OUT)]

    v0 = [None, p[0], p[1], p[3], p[6]]
    d0 = 1.0 + p[0] * p[0] + p[1] * p[1] + p[3] * p[3] + p[6] * p[6]
    s0 = 2.0 / d0
    u = [s0] + [s0 * v0[a] for a in range(1, _D)]
    q = [[None] * _D for _ in range(_D)]
    for a in range(_D):
        q[a][0] = (1.0 - u[0]) if a == 0 else (-u[a])
        for b in range(1, _D):
            if a == b:
                q[a][b] = 1.0 - u[a] * v0[b]
            else:
                q[a][b] = -(u[a] * v0[b])

    for t, pars in ((1, (p[2], p[4], p[7])), (2, (p[5], p[8])), (3, (p[9],))):
        d = 1.0
        for a_p in pars:
            d = d + a_p * a_p
        st = 2.0 / d
        for a in range(_D):
            w = q[a][t]
            for m, a_p in enumerate(pars):
                w = w + q[a][t + 1 + m] * a_p
            sw = st * w
            q[a][t] = q[a][t] - sw
            for m, a_p in enumerate(pars):
                q[a][t + 1 + m] = q[a][t + 1 + m] - sw * a_p
    for a in range(_D):
        q[a][_D - 1] = -q[a][_D - 1]

    rows = [q[a][b].reshape(1, _B) for a in range(_D) for b in range(_D)]
    at_ref[...] = jnp.concatenate(rows, axis=0)


def _main(xr, ec, wt, cv):
    nnz = xr.shape[0]
    nb = nnz // _B
    return pl.pallas_call(
        _main_body,
        grid=(nb,),
        in_specs=[
            pl.BlockSpec((_B, _HID), lambda i: (i, jnp.int32(0))),
            pl.BlockSpec((_B, _HID), lambda i: (i, jnp.int32(0))),
            pl.BlockSpec((16, 4 * _HID), lambda i: (jnp.int32(0), jnp.int32(0))),
            pl.BlockSpec((_OUT, 1), lambda i: (jnp.int32(0), jnp.int32(0))),
        ],
        out_specs=pl.BlockSpec((_D * _D, _B), lambda i: (jnp.int32(0), i)),
        out_shape=jax.ShapeDtypeStruct((_D * _D, nnz), jnp.float32),
    )(xr, ec, wt, cv)


def kernel(x, e, hyperedge_index, node_types, hyperedge_types,
           ln_gamma, ln_beta, W, b):
    n_nodes = x.shape[0] // _D
    n_edges = e.shape[0] // _D
    nnz = hyperedge_index.shape[1]

    x2 = x.reshape(n_nodes, _D * _HID)
    e2 = e.reshape(n_edges, _D * _HID)
    xm, em = _pool(x2, e2)

    row32 = hyperedge_index[0].astype(jnp.int32)
    col32 = hyperedge_index[1].astype(jnp.int32)
    xr, ec, hx = _make_gather(nnz)(xm, em, row32, col32)

    gw = W.astype(jnp.float32) * ln_gamma.astype(jnp.float32)[:, None]
    wc = gw - jnp.sum(gw, axis=0)[None, :] * (1.0 / (2 * _HID))
    wt = jnp.zeros((16, 4 * _HID), jnp.float32)
    wt = wt.at[0:_OUT, 0:2 * _HID].set(wc.T)
    wt = wt.at[10, 0:2 * _HID].set(1.0 / (2 * _HID))
    wt = wt.at[11, 2 * _HID:4 * _HID].set(1.0 / (2 * _HID))
    cv = (ln_beta.astype(jnp.float32) @ W.astype(jnp.float32)
          + b.astype(jnp.float32)).reshape(_OUT, 1)

    atT = _main(xr, ec, wt, cv)

    attrs = atT.T.reshape(-1).astype(jnp.float64)
    nlin = nnz * _D * _D
    h_index = (hx.reshape(nlin // 128, 2, 128).transpose(1, 0, 2)
               .reshape(2, nlin).astype(jnp.int64))
    return h_index, attrs


# speedup vs baseline: 389.3397x; 1.0576x over previous
import functools

import jax
import jax.numpy as jnp
from jax import lax
from jax.experimental import pallas as pl
from jax.experimental.pallas import tpu as pltpu
from jax.experimental.pallas import tpu_sc as plsc

_D = 5
_HID = 64
_OUT = _D * (_D - 1) // 2

_B = 2560
_R = _B // 128


def _pool_body(x_ref, e_ref, xm_ref, em_ref):
    xv = x_ref[...]
    ev = e_ref[...]
    xs = xv[:, 0:_HID]
    es = ev[:, 0:_HID]
    for d in range(1, _D):
        xs = xs + xv[:, d * _HID:(d + 1) * _HID]
        es = es + ev[:, d * _HID:(d + 1) * _HID]
    xm_ref[...] = xs * (1.0 / _D)
    em_ref[...] = es * (1.0 / _D)


def _pool(x2, e2):
    n = x2.shape[0]
    blk = 2000
    return pl.pallas_call(
        _pool_body,
        grid=(n // blk,),
        in_specs=[
            pl.BlockSpec((blk, _D * _HID), lambda i: (i, jnp.int32(0))),
            pl.BlockSpec((blk, _D * _HID), lambda i: (i, jnp.int32(0))),
        ],
        out_specs=[
            pl.BlockSpec((blk, _HID), lambda i: (i, jnp.int32(0))),
            pl.BlockSpec((blk, _HID), lambda i: (i, jnp.int32(0))),
        ],
        out_shape=[
            jax.ShapeDtypeStruct((n, _HID), jnp.float32),
            jax.ShapeDtypeStruct((n, _HID), jnp.float32),
        ],
    )(x2, e2)


def _make_gather(nnz):
    info = plsc.get_sparse_core_info()
    nc, ns = info.num_cores, info.num_subcores
    nw = nc * ns
    b_per_w = nnz // nw
    chunk = 1000
    nchunks = b_per_w // chunk
    mesh = plsc.VectorSubcoreMesh(core_axis_name="c", subcore_axis_name="s")

    @functools.partial(
        pl.kernel,
        out_type=[jax.ShapeDtypeStruct((nnz, _HID), jnp.float32)] * 2,
        mesh=mesh,
        scratch_types=[
            pltpu.VMEM((chunk + 16,), jnp.int32),
            pltpu.VMEM((chunk + 16,), jnp.int32),
            pltpu.VMEM((chunk, _HID), jnp.float32),
            pltpu.SemaphoreType.DMA,
        ],
        compiler_params=pltpu.CompilerParams(use_tc_tiling_on_sc=False, needs_layout_passes=False),
    )
    def gat(xm_hbm, em_hbm, row_hbm, col_hbm, xr_hbm, ec_hbm,
            idxr_v, idxc_v, rows_v, sem):
        wid = lax.axis_index("s") * nc + lax.axis_index("c")
        base = wid * b_per_w

        for j in range(nchunks):
            off = base + j * chunk
            pltpu.sync_copy(row_hbm.at[pl.ds(off, chunk)],
                            idxr_v.at[pl.ds(0, chunk)])
            pltpu.sync_copy(col_hbm.at[pl.ds(off, chunk)],
                            idxc_v.at[pl.ds(0, chunk)])
            pltpu.async_copy(xm_hbm.at[idxr_v.at[pl.ds(0, chunk)]],
                             rows_v, sem).wait()
            pltpu.sync_copy(rows_v, xr_hbm.at[pl.ds(off, chunk)])
            pltpu.async_copy(em_hbm.at[idxc_v.at[pl.ds(0, chunk)]],
                             rows_v, sem).wait()
            pltpu.sync_copy(rows_v, ec_hbm.at[pl.ds(off, chunk)])

    return gat


def _make_expand(nnz):
    info = plsc.get_sparse_core_info()
    nc, ns = info.num_cores, info.num_subcores
    nw = nc * ns
    mesh = plsc.VectorSubcoreMesh(core_axis_name="c", subcore_axis_name="s")
    ef = _D * _D
    npb = 512
    gpb = npb * ef
    nbat = nnz // npb
    bat_per_w = (nbat + nw - 1) // nw

    @functools.partial(
        pl.kernel,
        out_type=jax.ShapeDtypeStruct((2 * nnz * ef,), jnp.int32),
        mesh=mesh,
        scratch_types=[
            pltpu.VMEM((npb + 16,), jnp.int32),
            pltpu.VMEM((npb + 16,), jnp.int32),
            pltpu.VMEM((2 * gpb,), jnp.int32),
        ],
        compiler_params=pltpu.CompilerParams(use_tc_tiling_on_sc=False, needs_layout_passes=False),
    )
    def expand(row_hbm, col_hbm, hx_hbm, eidr_v, eidc_v, out_v):
        wid = lax.axis_index("s") * nc + lax.axis_index("c")
        lane = lax.iota(jnp.int32, 16)

        def bat(t, carry):
            b = wid + nw * t

            @pl.when(b < nbat)
            def _():
                n0 = b * npb
                pltpu.sync_copy(row_hbm.at[pl.ds(n0, npb)],
                                eidr_v.at[pl.ds(0, npb)])
                pltpu.sync_copy(col_hbm.at[pl.ds(n0, npb)],
                                eidc_v.at[pl.ds(0, npb)])

                def estep(s, c2):
                    l = s * 16 + lane
                    n = lax.shift_right_logical(l * 5243, jnp.int32(17))
                    k = l - 25 * n
                    ii = lax.shift_right_logical(k * 52, jnp.int32(8))
                    jj = k - 5 * ii
                    gr = plsc.load_gather(eidr_v, [n])
                    gc = plsc.load_gather(eidc_v, [n])
                    doff = 256 * lax.shift_right_logical(s, jnp.int32(3)) \
                        + 16 * (s & 7)
                    out_v[pl.ds(doff, 16)] = 5 * gr + ii
                    out_v[pl.ds(doff + 128, 16)] = 5 * gc + jj
                    return c2

                lax.fori_loop(jnp.int32(0), jnp.int32(gpb // 16), estep,
                              jnp.int32(0))
                pltpu.sync_copy(out_v, hx_hbm.at[pl.ds(b * 2 * gpb, 2 * gpb)])

            return carry

        lax.fori_loop(jnp.int32(0), jnp.int32(bat_per_w), bat, jnp.int32(0))

    return expand


def _to_rl(row_1b):
    return row_1b.reshape(_R, 128)


def _main_body(xr_ref, ec_ref, wt_ref, cv_ref, at_ref):
    f32 = jnp.float32
    xr = xr_ref[...]
    ec = ec_ref[...]
    hh = jnp.concatenate([xr, ec, xr * xr, ec * ec], axis=1)
    s = lax.dot_general(wt_ref[...], hh, (((1,), (1,)), ((), ())),
                        preferred_element_type=f32)
    mu = s[10:11, :]
    m2 = s[11:12, :]
    inv = lax.rsqrt(m2 - mu * mu + 1e-5)
    z = s[0:_OUT, :] * inv + cv_ref[...]
    sg = 1.0 / (1.0 + jnp.exp(-z))
    p = [_to_rl(sg[k:k + 1, :]) for k in range(_OUT)]

    v0 = [None, p[0], p[1], p[3], p[6]]
    d0 = 1.0 + p[0] * p[0] + p[1] * p[1] + p[3] * p[3] + p[6] * p[6]
    s0 = 2.0 / d0
    u = [s0] + [s0 * v0[a] for a in range(1, _D)]
    q = [[None] * _D for _ in range(_D)]
    for a in range(_D):
        q[a][0] = (1.0 - u[0]) if a == 0 else (-u[a])
        for b in range(1, _D):
            if a == b:
                q[a][b] = 1.0 - u[a] * v0[b]
            else:
                q[a][b] = -(u[a] * v0[b])

    for t, pars in ((1, (p[2], p[4], p[7])), (2, (p[5], p[8])), (3, (p[9],))):
        d = 1.0
        for a_p in pars:
            d = d + a_p * a_p
        st = 2.0 / d
        for a in range(_D):
            w = q[a][t]
            for m, a_p in enumerate(pars):
                w = w + q[a][t + 1 + m] * a_p
            sw = st * w
            q[a][t] = q[a][t] - sw
            for m, a_p in enumerate(pars):
                q[a][t + 1 + m] = q[a][t + 1 + m] - sw * a_p
    for a in range(_D):
        q[a][_D - 1] = -q[a][_D - 1]

    rows = [q[a][b].reshape(1, _B) for a in range(_D) for b in range(_D)]
    at_ref[...] = jnp.concatenate(rows, axis=0)


def _main(xr, ec, wt, cv):
    nnz = xr.shape[0]
    nb = nnz // _B
    return pl.pallas_call(
        _main_body,
        grid=(nb,),
        in_specs=[
            pl.BlockSpec((_B, _HID), lambda i: (i, jnp.int32(0))),
            pl.BlockSpec((_B, _HID), lambda i: (i, jnp.int32(0))),
            pl.BlockSpec((16, 4 * _HID), lambda i: (jnp.int32(0), jnp.int32(0))),
            pl.BlockSpec((_OUT, 1), lambda i: (jnp.int32(0), jnp.int32(0))),
        ],
        out_specs=pl.BlockSpec((_D * _D, _B), lambda i: (jnp.int32(0), i)),
        out_shape=jax.ShapeDtypeStruct((_D * _D, nnz), jnp.float32),
    )(xr, ec, wt, cv)


def kernel(x, e, hyperedge_index, node_types, hyperedge_types,
           ln_gamma, ln_beta, W, b):
    n_nodes = x.shape[0] // _D
    n_edges = e.shape[0] // _D
    nnz = hyperedge_index.shape[1]

    x2 = x.reshape(n_nodes, _D * _HID)
    e2 = e.reshape(n_edges, _D * _HID)
    xm, em = _pool(x2, e2)

    row32 = hyperedge_index[0].astype(jnp.int32)
    col32 = hyperedge_index[1].astype(jnp.int32)
    hx = _make_expand(nnz)(row32, col32)
    xr, ec = _make_gather(nnz)(xm, em, row32, col32)

    gw = W.astype(jnp.float32) * ln_gamma.astype(jnp.float32)[:, None]
    wc = gw - jnp.sum(gw, axis=0)[None, :] * (1.0 / (2 * _HID))
    wt = jnp.zeros((16, 4 * _HID), jnp.float32)
    wt = wt.at[0:_OUT, 0:2 * _HID].set(wc.T)
    wt = wt.at[10, 0:2 * _HID].set(1.0 / (2 * _HID))
    wt = wt.at[11, 2 * _HID:4 * _HID].set(1.0 / (2 * _HID))
    cv = (ln_beta.astype(jnp.float32) @ W.astype(jnp.float32)
          + b.astype(jnp.float32)).reshape(_OUT, 1)

    atT = _main(xr, ec, wt, cv)

    attrs = atT.T.reshape(-1).astype(jnp.float64)
    nlin = nnz * _D * _D
    h_index = (hx.reshape(nlin // 128, 2, 128).transpose(1, 0, 2)
               .reshape(2, nlin).astype(jnp.int64))
    return h_index, attrs


# speedup vs baseline: 406.4718x; 1.0440x over previous
import functools

import jax
import jax.numpy as jnp
from jax import lax
from jax.experimental import pallas as pl
from jax.experimental.pallas import tpu as pltpu
from jax.experimental.pallas import tpu_sc as plsc

_D = 5
_HID = 64
_OUT = _D * (_D - 1) // 2

_B = 2560
_R = _B // 128


def _pool_body(x_ref, e_ref, xm_ref, em_ref):
    xv = x_ref[...]
    ev = e_ref[...]
    xs = xv[:, 0:_HID]
    es = ev[:, 0:_HID]
    for d in range(1, _D):
        xs = xs + xv[:, d * _HID:(d + 1) * _HID]
        es = es + ev[:, d * _HID:(d + 1) * _HID]
    xm_ref[...] = xs * (1.0 / _D)
    em_ref[...] = es * (1.0 / _D)


def _pool(x2, e2):
    n = x2.shape[0]
    blk = 2000
    return pl.pallas_call(
        _pool_body,
        grid=(n // blk,),
        in_specs=[
            pl.BlockSpec((blk, _D * _HID), lambda i: (i, jnp.int32(0))),
            pl.BlockSpec((blk, _D * _HID), lambda i: (i, jnp.int32(0))),
        ],
        out_specs=[
            pl.BlockSpec((blk, _HID), lambda i: (i, jnp.int32(0))),
            pl.BlockSpec((blk, _HID), lambda i: (i, jnp.int32(0))),
        ],
        out_shape=[
            jax.ShapeDtypeStruct((n, _HID), jnp.float32),
            jax.ShapeDtypeStruct((n, _HID), jnp.float32),
        ],
    )(x2, e2)


def _make_gather(nnz):
    info = plsc.get_sparse_core_info()
    nc, ns = info.num_cores, info.num_subcores
    nw = nc * ns
    b_per_w = nnz // nw
    chunk = 400
    nchunks = b_per_w // chunk
    mesh = plsc.VectorSubcoreMesh(core_axis_name="c", subcore_axis_name="s")

    @functools.partial(
        pl.kernel,
        out_type=[jax.ShapeDtypeStruct((nnz, 2 * _HID), jnp.float32)] * 2,
        mesh=mesh,
        scratch_types=[
            pltpu.VMEM((chunk + 16,), jnp.int32),
            pltpu.VMEM((chunk + 16,), jnp.int32),
            pltpu.VMEM((chunk, 2 * _HID), jnp.float32),
            pltpu.SemaphoreType.DMA,
        ],
        compiler_params=pltpu.CompilerParams(use_tc_tiling_on_sc=True, needs_layout_passes=False),
    )
    def gat(t_hbm, row_hbm, col_hbm, xr_hbm, ec_hbm,
            idxr_v, idxc_v, rows_v, sem):
        wid = lax.axis_index("s") * nc + lax.axis_index("c")
        base = wid * b_per_w

        for j in range(nchunks):
            off = base + j * chunk
            pltpu.sync_copy(row_hbm.at[pl.ds(off, chunk)],
                            idxr_v.at[pl.ds(0, chunk)])
            pltpu.sync_copy(col_hbm.at[pl.ds(off, chunk)],
                            idxc_v.at[pl.ds(0, chunk)])
            pltpu.async_copy(t_hbm.at[idxr_v.at[pl.ds(0, chunk)]],
                             rows_v, sem).wait()
            pltpu.sync_copy(rows_v, xr_hbm.at[pl.ds(off, chunk)])
            pltpu.async_copy(t_hbm.at[idxc_v.at[pl.ds(0, chunk)]],
                             rows_v, sem).wait()
            pltpu.sync_copy(rows_v, ec_hbm.at[pl.ds(off, chunk)])

    return gat


def _make_expand(nnz):
    info = plsc.get_sparse_core_info()
    nc, ns = info.num_cores, info.num_subcores
    nw = nc * ns
    mesh = plsc.VectorSubcoreMesh(core_axis_name="c", subcore_axis_name="s")
    ef = _D * _D
    npb = 512
    gpb = npb * ef
    nbat = nnz // npb
    bat_per_w = (nbat + nw - 1) // nw

    @functools.partial(
        pl.kernel,
        out_type=jax.ShapeDtypeStruct((2 * nnz * ef,), jnp.int32),
        mesh=mesh,
        scratch_types=[
            pltpu.VMEM((npb + 16,), jnp.int32),
            pltpu.VMEM((npb + 16,), jnp.int32),
            pltpu.VMEM((2 * gpb,), jnp.int32),
        ],
        compiler_params=pltpu.CompilerParams(use_tc_tiling_on_sc=False, needs_layout_passes=False),
    )
    def expand(row_hbm, col_hbm, hx_hbm, eidr_v, eidc_v, out_v):
        wid = lax.axis_index("s") * nc + lax.axis_index("c")
        lane = lax.iota(jnp.int32, 16)

        def bat(t, carry):
            b = wid + nw * t

            @pl.when(b < nbat)
            def _():
                n0 = b * npb
                pltpu.sync_copy(row_hbm.at[pl.ds(n0, npb)],
                                eidr_v.at[pl.ds(0, npb)])
                pltpu.sync_copy(col_hbm.at[pl.ds(n0, npb)],
                                eidc_v.at[pl.ds(0, npb)])

                def estep(s, c2):
                    l = s * 16 + lane
                    n = lax.shift_right_logical(l * 5243, jnp.int32(17))
                    k = l - 25 * n
                    ii = lax.shift_right_logical(k * 52, jnp.int32(8))
                    jj = k - 5 * ii
                    gr = plsc.load_gather(eidr_v, [n])
                    gc = plsc.load_gather(eidc_v, [n])
                    doff = 256 * lax.shift_right_logical(s, jnp.int32(3)) \
                        + 16 * (s & 7)
                    out_v[pl.ds(doff, 16)] = 5 * gr + ii
                    out_v[pl.ds(doff + 128, 16)] = 5 * gc + jj
                    return c2

                lax.fori_loop(jnp.int32(0), jnp.int32(gpb // 16), estep,
                              jnp.int32(0))
                pltpu.sync_copy(out_v, hx_hbm.at[pl.ds(b * 2 * gpb, 2 * gpb)])

            return carry

        lax.fori_loop(jnp.int32(0), jnp.int32(bat_per_w), bat, jnp.int32(0))

    return expand


def _to_rl(row_1b):
    return row_1b.reshape(_R, 128)


def _main_body(xr_ref, ec_ref, wt_ref, cv_ref, at_ref):
    f32 = jnp.float32
    xr = xr_ref[:, 0:_HID]
    ec = ec_ref[:, _HID:2 * _HID]
    hh = jnp.concatenate([xr, ec, xr * xr, ec * ec], axis=1)
    s = lax.dot_general(wt_ref[...], hh, (((1,), (1,)), ((), ())),
                        preferred_element_type=f32)
    mu = s[10:11, :]
    m2 = s[11:12, :]
    inv = lax.rsqrt(m2 - mu * mu + 1e-5)
    z = s[0:_OUT, :] * inv + cv_ref[...]
    sg = 1.0 / (1.0 + jnp.exp(-z))
    p = [_to_rl(sg[k:k + 1, :]) for k in range(_OUT)]

    v0 = [None, p[0], p[1], p[3], p[6]]
    d0 = 1.0 + p[0] * p[0] + p[1] * p[1] + p[3] * p[3] + p[6] * p[6]
    s0 = 2.0 / d0
    u = [s0] + [s0 * v0[a] for a in range(1, _D)]
    q = [[None] * _D for _ in range(_D)]
    for a in range(_D):
        q[a][0] = (1.0 - u[0]) if a == 0 else (-u[a])
        for b in range(1, _D):
            if a == b:
                q[a][b] = 1.0 - u[a] * v0[b]
            else:
                q[a][b] = -(u[a] * v0[b])

    for t, pars in ((1, (p[2], p[4], p[7])), (2, (p[5], p[8])), (3, (p[9],))):
        d = 1.0
        for a_p in pars:
            d = d + a_p * a_p
        st = 2.0 / d
        for a in range(_D):
            w = q[a][t]
            for m, a_p in enumerate(pars):
                w = w + q[a][t + 1 + m] * a_p
            sw = st * w
            q[a][t] = q[a][t] - sw
            for m, a_p in enumerate(pars):
                q[a][t + 1 + m] = q[a][t + 1 + m] - sw * a_p
    for a in range(_D):
        q[a][_D - 1] = -q[a][_D - 1]

    rows = [q[a][b].reshape(1, _B) for a in range(_D) for b in range(_D)]
    at_ref[...] = jnp.concatenate(rows, axis=0)


def _main(xr, ec, wt, cv):
    nnz = xr.shape[0]
    nb = nnz // _B
    return pl.pallas_call(
        _main_body,
        grid=(nb,),
        in_specs=[
            pl.BlockSpec((_B, 2 * _HID), lambda i: (i, jnp.int32(0))),
            pl.BlockSpec((_B, 2 * _HID), lambda i: (i, jnp.int32(0))),
            pl.BlockSpec((16, 4 * _HID), lambda i: (jnp.int32(0), jnp.int32(0))),
            pl.BlockSpec((_OUT, 1), lambda i: (jnp.int32(0), jnp.int32(0))),
        ],
        out_specs=pl.BlockSpec((_D * _D, _B), lambda i: (jnp.int32(0), i)),
        out_shape=jax.ShapeDtypeStruct((_D * _D, nnz), jnp.float32),
    )(xr, ec, wt, cv)


def kernel(x, e, hyperedge_index, node_types, hyperedge_types,
           ln_gamma, ln_beta, W, b):
    n_nodes = x.shape[0] // _D
    n_edges = e.shape[0] // _D
    nnz = hyperedge_index.shape[1]

    x2 = x.reshape(n_nodes, _D * _HID)
    e2 = e.reshape(n_edges, _D * _HID)
    xm, em = _pool(x2, e2)

    row32 = hyperedge_index[0].astype(jnp.int32)
    col32 = hyperedge_index[1].astype(jnp.int32)
    hx = _make_expand(nnz)(row32, col32)
    tbl = jnp.concatenate([xm, em], axis=1)
    xr, ec = _make_gather(nnz)(tbl, row32, col32)

    gw = W.astype(jnp.float32) * ln_gamma.astype(jnp.float32)[:, None]
    wc = gw - jnp.sum(gw, axis=0)[None, :] * (1.0 / (2 * _HID))
    wt = jnp.zeros((16, 4 * _HID), jnp.float32)
    wt = wt.at[0:_OUT, 0:2 * _HID].set(wc.T)
    wt = wt.at[10, 0:2 * _HID].set(1.0 / (2 * _HID))
    wt = wt.at[11, 2 * _HID:4 * _HID].set(1.0 / (2 * _HID))
    cv = (ln_beta.astype(jnp.float32) @ W.astype(jnp.float32)
          + b.astype(jnp.float32)).reshape(_OUT, 1)

    atT = _main(xr, ec, wt, cv)

    attrs = atT.T.reshape(-1).astype(jnp.float64)
    nlin = nnz * _D * _D
    h_index = (hx.reshape(nlin // 128, 2, 128).transpose(1, 0, 2)
               .reshape(2, nlin).astype(jnp.int64))
    return h_index, attrs
